# baseline (speedup 1.0000x reference)
import functools

import jax
import jax.numpy as jnp
from jax import lax
from jax.experimental import pallas as pl
from jax.experimental.pallas import tpu as pltpu
from jax.experimental.pallas import tpu_sc as plsc

N = 1048576
BLK = 4096
G = N // BLK
R2 = BLK // 16
SLAB = 1024
LADDER = (16384, 16384, 8192, 4096, 2048, 1024, 512, 256, 128)


def _k1_body(x_ref, pref_ref, thr_ref, acc_ref):
    pid = pl.program_id(0)

    @pl.when(pid == 0)
    def _():
        acc_ref[0] = 0
        acc_ref[1] = 0

    lanes = lax.broadcasted_iota(jnp.int32, (R2, 128), 1)
    qm = (lanes & 7) == 2
    xb = x_ref[...]
    n1 = jnp.sum(((xb == 1.0) & qm).astype(jnp.int32))
    n2 = jnp.sum(((xb == 2.0) & qm).astype(jnp.int32))
    a1 = acc_ref[0]
    a2 = acc_ref[1]

    c16 = lax.broadcasted_iota(jnp.int32, (1, 16), 1)
    pref_ref[...] = jnp.where(c16 == 0, a1,
                              jnp.where(c16 == 1, a2, 0))[None]

    a1n = a1 + n1
    a2n = a2 + n2
    c0 = N - a1n - a2n
    thr_ref[...] = jnp.where(c16 == 0, c0,
                             jnp.where(c16 == 1, c0 + a1n, 0))
    acc_ref[0] = a1n
    acc_ref[1] = a2n


def _k1(x2d):
    return pl.pallas_call(
        _k1_body,
        grid=(G,),
        in_specs=[pl.BlockSpec((R2, 128), lambda i: (i, 0))],
        out_specs=[
            pl.BlockSpec((1, 1, 16), lambda i: (i, 0, 0)),
            pl.BlockSpec((1, 16), lambda i: (0, 0)),
        ],
        out_shape=[
            jax.ShapeDtypeStruct((G, 1, 16), jnp.int32),
            jax.ShapeDtypeStruct((1, 16), jnp.int32),
        ],
        scratch_shapes=[pltpu.SMEM((2,), jnp.int32)],
        compiler_params=pltpu.CompilerParams(
            dimension_semantics=("arbitrary",)),
    )(x2d)


def _s2(x2d, pref, thr):
    info = plsc.get_sparse_core_info()
    nc, ns = info.num_cores, info.num_subcores
    nw = nc * ns
    m = N // nw
    rows2d = SLAB // 16
    nslab = m // SLAB
    bufsz = m + 256
    mesh = plsc.VectorSubcoreMesh(core_axis_name="c", subcore_axis_name="s")

    @functools.partial(
        pl.kernel,
        mesh=mesh,
        out_type=jax.ShapeDtypeStruct((N + 128,), jnp.float32),
        scratch_types=[
            pltpu.VMEM((rows2d, 128), jnp.float32),
            pltpu.VMEM((bufsz,), jnp.float32),
            pltpu.VMEM((bufsz,), jnp.float32),
            pltpu.VMEM((bufsz,), jnp.float32),
            pltpu.VMEM((6, 128), jnp.int32),
            pltpu.VMEM((16,), jnp.int32),
            pltpu.VMEM((16,), jnp.int32),
            pltpu.SemaphoreType.DMA,
        ],
        compiler_params=pltpu.CompilerParams(needs_layout_passes=False),
    )
    def s2(x_h, pref_h, thr_h, ws_h, xv, b0, b1, b2, idxb, prefv, thrv, sem):
        wid = lax.axis_index("s") * nc + lax.axis_index("c")
        lane = lax.iota(jnp.int32, 16)
        z = lane * 0

        pltpu.sync_copy(pref_h.at[(m // BLK) * wid], prefv)
        pltpu.sync_copy(thr_h.at[0], thrv)
        p = prefv[...]
        t = thrv[...]
        pre1 = jnp.sum(jnp.where(lane == 0, p, 0))
        pre2 = jnp.sum(jnp.where(lane == 1, p, 0))
        c0 = jnp.sum(jnp.where(lane == 0, t, 0))
        c01 = jnp.sum(jnp.where(lane == 1, t, 0))

        g0 = wid * m - pre1 - pre2
        g1 = c0 + pre1
        g2 = c01 + pre2
        bo0 = g0 & 127
        bo1 = g1 & 127
        bo2 = g2 & 127

        def slab_body(sidx, carry):
            p0, p1, p2 = carry
            row0 = pl.multiple_of((wid * m + sidx * SLAB) // 16, 8)
            pltpu.sync_copy(x_h.at[pl.ds(row0, rows2d)], xv)

            def grp_body(g, carry):
                p0, p1, p2 = carry
                col = lane * 8
                wf = plsc.load_gather(xv, [z + g, col])
                qf = plsc.load_gather(xv, [z + g, col + 2])
                m1 = qf == 1.0
                m2 = qf == 2.0
                m0 = qf == 0.0
                plsc.store_compressed(b0.at[pl.ds(p0, 16)], wf, mask=m0)
                plsc.store_compressed(b1.at[pl.ds(p1, 16)], wf, mask=m1)
                plsc.store_compressed(b2.at[pl.ds(p2, 16)], wf, mask=m2)
                d1 = jnp.sum(m1.astype(jnp.int32))
                d2 = jnp.sum(m2.astype(jnp.int32))
                return (p0 + (16 - d1 - d2), p1 + d1, p2 + d2)

            return lax.fori_loop(0, SLAB // 16, grp_body, (p0, p1, p2))

        p0, p1, p2 = lax.fori_loop(0, nslab, slab_body, (bo0, bo1, bo2))

        def flush(buf, bo, gk, pk, bslot):
            gk0 = gk - bo
            hhi = jnp.minimum(pk, 128)
            nbhi = pk >> 7
            tstart = nbhi * 128
            tlo = jnp.where(nbhi > 0, 0, 128)
            thi = jnp.where(nbhi > 0, pk - tstart, 0)

            def fill(slot, lo, hi, base):
                def u_body(u, _):
                    lv = u * 16 + lane
                    keep = (lv >= lo) & (lv < hi)
                    iv = jnp.where(keep, base + lv, N + lv)
                    plsc.store_scatter(idxb, [z + slot, lv], iv)
                    return 0
                lax.fori_loop(0, 8, u_body, 0)

            fill(2 * bslot, bo, hhi, gk0)
            fill(2 * bslot + 1, tlo, thi, gk0 + tstart)
            cp_h = pltpu.async_copy(buf.at[pl.ds(0, 128)],
                                    ws_h.at[idxb.at[2 * bslot]], sem)
            ts = pl.multiple_of(tstart, 8)
            cp_t = pltpu.async_copy(buf.at[pl.ds(ts, 128)],
                                    ws_h.at[idxb.at[2 * bslot + 1]], sem)
            cp_h.wait()
            cp_t.wait()

            rem = jnp.maximum(nbhi - 1, 0) * 128
            off = jnp.int32(128)
            for sz in LADDER:
                cond = rem >= sz

                @pl.when(cond)
                def _(off=off, sz=sz):
                    o = pl.multiple_of(off, 8)
                    d = pl.multiple_of(gk0 + o, 8)
                    pltpu.async_copy(buf.at[pl.ds(o, sz)],
                                     ws_h.at[pl.ds(d, sz)], sem).wait()

                off = jnp.where(cond, off + sz, off)
                rem = jnp.where(cond, rem - sz, rem)

        flush(b0, bo0, g0, p0, 0)
        flush(b1, bo1, g1, p1, 1)
        flush(b2, bo2, g2, p2, 2)

    return s2(x2d, pref, thr)


def _s3(x2d, ws, thr):
    info = plsc.get_sparse_core_info()
    nc, ns = info.num_cores, info.num_subcores
    nw = nc * ns
    m = N // nw
    rows2d = SLAB // 16
    nslab = m // SLAB
    mesh = plsc.VectorSubcoreMesh(core_axis_name="c", subcore_axis_name="s")

    @functools.partial(
        pl.kernel,
        mesh=mesh,
        out_type=jax.ShapeDtypeStruct((8 * N,), jnp.float32),
        scratch_types=[
            pltpu.VMEM((rows2d, 128), jnp.float32),
            pltpu.VMEM((SLAB,), jnp.float32),
            pltpu.VMEM((8 * SLAB,), jnp.float32),
            pltpu.VMEM((16,), jnp.int32),
            pltpu.SemaphoreType.DMA,
        ],
        compiler_params=pltpu.CompilerParams(needs_layout_passes=False),
    )
    def s3(x_h, ws_h, thr_h, o_h, xv, wsv, ov, thrv, sem):
        wid = lax.axis_index("s") * nc + lax.axis_index("c")
        lane = lax.iota(jnp.int32, 16)
        z = lane * 0

        pltpu.sync_copy(thr_h.at[0], thrv)
        t = thrv[...]
        c0 = jnp.sum(jnp.where(lane == 0, t, 0))
        c01 = jnp.sum(jnp.where(lane == 1, t, 0))

        def slab_body(sidx, _):
            base = wid * m + sidx * SLAB
            row0 = pl.multiple_of(base // 16, 8)
            pltpu.sync_copy(x_h.at[pl.ds(row0, rows2d)], xv)
            bws = pl.multiple_of(base, 8)
            pltpu.sync_copy(ws_h.at[pl.ds(bws, SLAB)], wsv)

            def grp_body(g, _):
                col = lane * 8
                wf = plsc.load_gather(xv, [z + g, col])
                wsg = wsv[pl.ds(g * 16, 16)]
                j = base + g * 16 + lane
                in0 = j < c0
                in1 = j < c01
                dw = jnp.where(in0, 0.0,
                               jnp.where(in1, 0.3465 * wsg, 0.5))
                dsv = jnp.where(in0, 0.0, wsg)
                flat = g * 128 + col
                plsc.store_scatter(ov, [flat], dw)
                plsc.store_scatter(ov, [flat + 1], dsv)
                plsc.store_scatter(ov, [flat + 2], z.astype(jnp.float32))
                plsc.store_scatter(ov, [flat + 3],
                                   z + jnp.float32(1.0 / 3))
                w20 = wf / 20
                w05 = 0.05 * wf
                plsc.store_scatter(ov, [flat + 4], w20)
                plsc.store_scatter(ov, [flat + 5], w05)
                plsc.store_scatter(ov, [flat + 6], w05)
                plsc.store_scatter(ov, [flat + 7], w20)
                return 0

            lax.fori_loop(0, SLAB // 16, grp_body, 0)
            ob = pl.multiple_of(base * 8, 8)
            pltpu.sync_copy(ov, o_h.at[pl.ds(ob, 8 * SLAB)])
            return 0

        lax.fori_loop(0, nslab, slab_body, 0)

    return s3(x2d, ws, thr)


def kernel(t, x):
    x2d = jnp.reshape(x, (N // 16, 128))
    pref, thr = _k1(x2d)
    ws = _s2(x2d, jnp.reshape(pref, (G, 16)), thr)
    out = _s3(x2d, ws, thr)
    return jnp.reshape(out, (N, 8))
